# baseline (speedup 1.0000x reference)
import functools

import jax
import jax.numpy as jnp
from jax import lax
from jax.experimental import pallas as pl
from jax.experimental.pallas import tpu as pltpu
from jax.experimental.pallas import tpu_sc as plsc

_NW = 32
_F = 1024
_IDXW = 128


def _build_m2_sc(id_reduce, kidx, m, nedges, kmax):
    nm, emb = m.shape
    slots = nedges * kmax
    assert slots % _NW == 0
    r = slots // _NW
    assert r % 16 == 0
    nchunk = 20
    assert nm % (nchunk * 16) == 0
    chunk = nm // nchunk
    assert chunk % 8 == 0
    zrows = 1000 if r % 1000 == 0 else 16
    assert r % zrows == 0 and zrows <= _F

    mesh = plsc.VectorSubcoreMesh(core_axis_name="c", subcore_axis_name="s",
                                  num_cores=2, num_subcores=16)

    @functools.partial(
        pl.kernel,
        out_type=jax.ShapeDtypeStruct((slots + 16, emb), jnp.float32),
        mesh=mesh,
        scratch_types=[
            pltpu.VMEM((chunk,), jnp.int32),
            pltpu.VMEM((chunk,), jnp.int32),
            pltpu.VMEM((r,), jnp.int32),
            pltpu.VMEM((_F,), jnp.int32),
            pltpu.VMEM((_F,), jnp.int32),
            pltpu.VMEM((_F // _IDXW, _IDXW), jnp.int32),
            pltpu.VMEM((_F // _IDXW, _IDXW), jnp.int32),
            pltpu.VMEM((_F, emb), jnp.float32),
            pltpu.SemaphoreType.DMA,
        ],
        compiler_params=pltpu.CompilerParams(needs_layout_passes=False,
                                             use_tc_tiling_on_sc=False),
    )
    def build(id_hbm, k_hbm, m_hbm, m2_hbm, idbuf, kbuf, winner, cj, cs,
              jidx2d, sidx2d, rows, sem):
        wid = lax.axis_index("c") * 16 + lax.axis_index("s")
        lo = wid * r
        lane = lax.iota(jnp.int32, 16)
        zeros16 = jnp.zeros((16,), jnp.int32)

        def initw(i, _):
            winner[pl.ds(i * 16, 16)] = zeros16
            return 0
        lax.fori_loop(0, r // 16, initw, 0)

        def chunk_body(ci, _):
            pltpu.sync_copy(id_hbm.at[pl.ds(ci * chunk, chunk)], idbuf)
            pltpu.sync_copy(k_hbm.at[pl.ds(ci * chunk, chunk)], kbuf)

            def vec_body(v, _):
                idv = idbuf[pl.ds(v * 16, 16)]
                kv = kbuf[pl.ds(v * 16, 16)]
                slot = idv * kmax + kv
                mask = (slot >= lo) & (slot < lo + r)
                local = jnp.where(mask, slot - lo, 0)
                jv = ci * chunk + v * 16 + lane
                plsc.store_scatter(winner, [local], jv + 1, mask=mask)
                return 0
            lax.fori_loop(0, chunk // 16, vec_body, 0)
            return 0
        lax.fori_loop(0, nchunk, chunk_body, 0)

        def zero_rows(i, _):
            rows[i, pl.ds(0, 16)] = jnp.zeros((16,), jnp.float32)
            rows[i, pl.ds(16, 16)] = jnp.zeros((16,), jnp.float32)
            return 0
        lax.fori_loop(0, zrows, zero_rows, 0)

        def memset_body(t, _):
            pltpu.sync_copy(rows.at[pl.ds(0, zrows)],
                            m2_hbm.at[pl.ds(lo + t * zrows, zrows)])
            return 0
        lax.fori_loop(0, r // zrows, memset_body, 0)

        def reset_defaults(i, _):
            cj[pl.ds(i * 16, 16)] = zeros16
            cs[pl.ds(i * 16, 16)] = slots + lane
            return 0
        lax.fori_loop(0, _F // 16, reset_defaults, 0)

        def flush(_):
            def cpidx(i, _):
                q = i // (_IDXW // 16)
                t = i % (_IDXW // 16)
                jidx2d[q, pl.ds(t * 16, 16)] = cj[pl.ds(i * 16, 16)]
                sidx2d[q, pl.ds(t * 16, 16)] = cs[pl.ds(i * 16, 16)]
                return 0
            lax.fori_loop(0, _F // 16, cpidx, 0)
            for q in range(_F // _IDXW):
                pltpu.async_copy(m_hbm.at[jidx2d.at[q]],
                                 rows.at[pl.ds(q * _IDXW, _IDXW)], sem).wait()
            for q in range(_F // _IDXW):
                pltpu.async_copy(rows.at[pl.ds(q * _IDXW, _IDXW)],
                                 m2_hbm.at[sidx2d.at[q]], sem).wait()
            lax.fori_loop(0, _F // 16, reset_defaults, 0)
            return jnp.int32(0)

        def sweep(v, cnt):
            w16 = winner[pl.ds(v * 16, 16)]
            valid = w16 > 0
            j16 = w16 - 1
            slotg = lo + v * 16 + lane
            plsc.store_compressed(cj.at[pl.ds(cnt, 16)], j16, mask=valid)
            plsc.store_compressed(cs.at[pl.ds(cnt, 16)], slotg, mask=valid)
            cnt = cnt + jnp.sum(valid.astype(jnp.int32))
            return lax.cond(cnt > _F - 16, flush, lambda c: c, cnt)
        cnt = lax.fori_loop(0, r // 16, sweep, jnp.int32(0))
        flush(cnt)

    return build(id_reduce, kidx, m)


def _tc_compute(a2, s2, m2r, w2, nedges, nsph, kmax, emb, interm, units):
    eblk = 800 if nedges % 800 == 0 else 8
    assert nedges % eblk == 0

    def body(a_ref, s_ref, m_ref, w_ref, o_ref, u_scr):
        a = a_ref[...]
        s = s_ref[...]
        mm = m_ref[...]
        sumk = []
        for si in range(nsph):
            acc = None
            for k in range(kmax):
                c0 = si * kmax + k
                t = s[:, c0:c0 + 1] * mm[:, k * emb:(k + 1) * emb]
                acc = t if acc is None else acc + t
            sumk.append(acc)
        for i in range(interm):
            acc = None
            for si in range(nsph):
                c0 = i * nsph + si
                t = a[:, c0:c0 + 1] * sumk[si]
                acc = t if acc is None else acc + t
            u_scr[:, i * emb:(i + 1) * emb] = acc
        o_ref[...] = jnp.dot(u_scr[...], w_ref[...],
                             preferred_element_type=jnp.float32)

    grid = (nedges // eblk,)
    return pl.pallas_call(
        body,
        grid=grid,
        in_specs=[
            pl.BlockSpec((eblk, interm * nsph), lambda i: (i, 0)),
            pl.BlockSpec((eblk, nsph * kmax), lambda i: (i, 0)),
            pl.BlockSpec((eblk, kmax * emb), lambda i: (i, 0)),
            pl.BlockSpec((interm * emb, units), lambda i: (0, 0)),
        ],
        out_specs=pl.BlockSpec((eblk, units), lambda i: (i, 0)),
        out_shape=jax.ShapeDtypeStruct((nedges, units), jnp.float32),
        scratch_shapes=[pltpu.VMEM((eblk, interm * emb), jnp.float32)],
        compiler_params=pltpu.CompilerParams(
            dimension_semantics=("arbitrary",)),
    )(a2, s2, m2r, w2)


def kernel(rbf_W1, sph, m, id_reduce, Kidx, weight):
    nedges, interm, nsph = rbf_W1.shape
    kmax = sph.shape[2]
    nm, emb = m.shape
    units = weight.shape[2]

    id32 = id_reduce.astype(jnp.int32)
    k32 = Kidx.astype(jnp.int32)

    m2 = _build_m2_sc(id32, k32, m, nedges, kmax)
    m2r = m2.reshape((nedges * kmax + 16) // kmax, kmax * emb)

    a2 = rbf_W1.reshape(nedges, interm * nsph)
    s2 = sph.reshape(nedges, nsph * kmax)
    w2 = jnp.transpose(weight, (1, 0, 2)).reshape(interm * emb, units)

    return _tc_compute(a2, s2, m2r, w2, nedges, nsph, kmax, emb, interm, units)
